# baseline (speedup 1.0000x reference)
import functools
import numpy as np
import jax
import jax.numpy as jnp
from jax import lax
from jax.experimental import pallas as pl
from jax.experimental.pallas import tpu as pltpu
from jax.experimental.pallas import tpu_sc as plsc

_MODEL_DIM = 64
_MAX_POS = 512


def _positional_encoding(position, model_dim):
    pos = np.arange(position)[:, np.newaxis].astype(np.float32)
    i = np.arange(model_dim)[np.newaxis, :].astype(np.float32)
    angle_rates = 1.0 / np.power(10000, 2 * (i // 2) / np.float32(model_dim))
    angle_rads = pos * angle_rates
    angle_rads[:, 0::2] = np.sin(angle_rads[:, 0::2])
    angle_rads[:, 1::2] = np.cos(angle_rads[:, 1::2])
    return angle_rads.astype(np.float32)


_POS_ENC = _positional_encoding(_MAX_POS, _MODEL_DIM)


@functools.partial(jax.jit, static_argnums=(3, 4, 5))
def _embed(idx_flat, table, pos, batch, seq, dim):
    B = batch * seq
    NC, NS = 2, 16
    NW = NC * NS
    rows_per_w = B // NW
    seqs_per_chunk = 2
    chunk = seqs_per_chunk * seq
    n_chunks = rows_per_w // chunk
    nvec = dim // 16
    scale = float(np.sqrt(dim))

    mesh = plsc.VectorSubcoreMesh(core_axis_name="c", subcore_axis_name="s")

    @functools.partial(
        pl.kernel,
        mesh=mesh,
        compiler_params=pltpu.CompilerParams(use_tc_tiling_on_sc=False),
        out_type=jax.ShapeDtypeStruct((B, dim), jnp.float32),
        scratch_types=[
            pltpu.VMEM((rows_per_w,), jnp.int32),
            pltpu.VMEM((chunk, dim), jnp.float32),
            pltpu.VMEM((chunk, dim), jnp.float32),
            pltpu.SemaphoreType.DMA,
        ],
    )
    def k(idx_hbm, table_hbm, pos_hbm, out_hbm, idx_v, rows_v, pos_v, sem):
        wid = lax.axis_index("s") * NC + lax.axis_index("c")
        base = wid * rows_per_w
        pltpu.sync_copy(idx_hbm.at[pl.ds(base, rows_per_w)], idx_v)
        for r in range(seqs_per_chunk):
            pltpu.sync_copy(pos_hbm, pos_v.at[pl.ds(r * seq, seq)])

        def chunk_body(c, carry):
            off = c * chunk
            pltpu.async_copy(
                table_hbm.at[idx_v.at[pl.ds(off, chunk)]], rows_v, sem
            ).wait()

            def row_body(i, carry2):
                for j in range(nvec):
                    sl = pl.ds(j * 16, 16)
                    rows_v[i, sl] = rows_v[i, sl] * scale + pos_v[i, sl]
                return carry2

            lax.fori_loop(0, chunk, row_body, 0)
            pltpu.sync_copy(rows_v, out_hbm.at[pl.ds(base + off, chunk)])
            return carry

        lax.fori_loop(0, n_chunks, chunk_body, 0)

    return k(idx_flat, table, pos)


def kernel(inp, table, training):
    batch, seq = inp.shape
    dim = table.shape[1]
    pos = jnp.asarray(_POS_ENC[:seq])
    out = _embed(inp.reshape(-1), table, pos, batch, seq, dim)
    return out.reshape(batch, seq, dim)
